# baseline (speedup 1.0000x reference)
import functools

import jax
import jax.numpy as jnp
from jax import lax
from jax.experimental import pallas as pl
from jax.experimental.pallas import tpu as pltpu
from jax.experimental.pallas import tpu_sc as plsc

EMB = 32
HID = 128
OUT = 2
SEQ = 200
BATCH = 4096

NC = 2
NS = 16
NW = NC * NS
BPW = BATCH // NW
LANES = 16
VPR = EMB // LANES
VECS = BPW * VPR


VOCAB = 1000000
VT = 128
NT = 7813
VOCAB_P = NT * VT
KPW = 246
TILE_ELEMS = VT * EMB


def _transpose_table(table_t):
    mesh = plsc.VectorSubcoreMesh(core_axis_name="c", subcore_axis_name="s")

    @functools.partial(
        pl.kernel,
        mesh=mesh,
        out_type=jax.ShapeDtypeStruct((VOCAB_P * EMB,), jnp.float32),
        scratch_types=[
            pltpu.VMEM((EMB, VT), jnp.float32),
            pltpu.VMEM((EMB, VT), jnp.float32),
            pltpu.VMEM((TILE_ELEMS,), jnp.float32),
            pltpu.VMEM((TILE_ELEMS,), jnp.float32),
            pltpu.SemaphoreType.DMA,
            pltpu.SemaphoreType.DMA,
            pltpu.SemaphoreType.DMA,
            pltpu.SemaphoreType.DMA,
        ],
        compiler_params=pltpu.CompilerParams(
            use_tc_tiling_on_sc=True, needs_layout_passes=False),
    )
    def tr(tt_hbm, out_hbm, src0, src1, dst0, dst1, si0, si1, so0, so1):
        wid = lax.axis_index("s") * NC + lax.axis_index("c")
        t0 = wid * KPW
        srcs = (src0, src1)
        dsts = (dst0, dst1)
        sin = (si0, si1)
        sout = (so0, so1)

        lane = lax.iota(jnp.int32, LANES)

        def voff(k):
            return jnp.minimum(t0 + k, NT - 1) * VT

        def start_in(k, b):
            pltpu.make_async_copy(
                tt_hbm.at[:, pl.ds(voff(k), VT)], srcs[b], sin[b]).start()

        def start_out(k, b):
            pltpu.make_async_copy(
                dsts[b], out_hbm.at[pl.ds(voff(k) * EMB, TILE_ELEMS)],
                sout[b]).start()

        def wait_in(b):
            pltpu.make_async_copy(
                tt_hbm.at[:, pl.ds(0, VT)], srcs[b], sin[b]).wait()

        def wait_out(b):
            pltpu.make_async_copy(
                dsts[b], out_hbm.at[pl.ds(0, TILE_ELEMS)], sout[b]).wait()

        def scatter(b):
            @plsc.parallel_loop(0, EMB * (VT // LANES), unroll=8)
            def _(i):
                d = ((i >> 3) + lane) & (EMB - 1)
                vj = (i & 7) * LANES + lane
                x = plsc.load_gather(srcs[b], [d, vj])
                plsc.store_scatter(dsts[b], [vj * EMB + d], x)

        start_in(0, 0)
        start_in(1, 1)

        def gbody(g, _):
            k0 = 2 * g

            @pl.when(g > 0)
            def _():
                wait_out(0)

            wait_in(0)
            scatter(0)
            start_out(k0, 0)

            @pl.when(k0 + 2 < KPW)
            def _():
                start_in(k0 + 2, 0)

            @pl.when(g > 0)
            def _():
                wait_out(1)

            wait_in(1)
            scatter(1)
            start_out(k0 + 1, 1)

            @pl.when(k0 + 3 < KPW)
            def _():
                start_in(k0 + 3, 1)

            return 0

        lax.fori_loop(0, KPW // 2, gbody, 0)
        wait_out(0)
        wait_out(1)

    return tr(table_t)


VC_TC = 13 * 128
NB_TC = VOCAB_P // VC_TC


def _transpose_table_tc(table_t):
    def body(in_ref, o_ref):
        x = in_ref[...]
        o_ref[...] = x.T.reshape(-1)

    return pl.pallas_call(
        body,
        grid=(NB_TC,),
        in_specs=[pl.BlockSpec((EMB, VC_TC), lambda i: (0, i))],
        out_specs=pl.BlockSpec((VC_TC * EMB,), lambda i: (i,)),
        out_shape=jax.ShapeDtypeStruct((VOCAB_P * EMB,), jnp.float32),
    )(table_t)


def _pool_sums(text, emb_table):
    mesh = plsc.VectorSubcoreMesh(core_axis_name="c", subcore_axis_name="s")

    @functools.partial(
        pl.kernel,
        mesh=mesh,
        out_type=jax.ShapeDtypeStruct((BATCH, EMB), jnp.float32),
        scratch_types=[
            pltpu.VMEM((SEQ, BPW), jnp.int32),
            pltpu.VMEM((BPW, EMB), jnp.float32),
            pltpu.VMEM((8, BPW, EMB), jnp.float32),
        ] + [pltpu.SemaphoreType.DMA] * 8,
        compiler_params=pltpu.CompilerParams(use_tc_tiling_on_sc=False),
    )
    def pool(text_hbm, table_hbm, out_hbm, idx_v, acc_v, rows_v, *sems):
        wid = lax.axis_index("s") * NC + lax.axis_index("c")
        base = wid * BPW
        pltpu.sync_copy(text_hbm.at[:, pl.ds(base, BPW)], idx_v)

        zero = jnp.zeros((LANES,), jnp.float32)

        def zbody(r, _):
            acc_v[r, pl.ds(0, LANES)] = zero
            acc_v[r, pl.ds(LANES, LANES)] = zero
            return 0

        lax.fori_loop(0, BPW, zbody, 0, unroll=8)

        def start(s, b):
            pltpu.make_async_copy(
                table_hbm.at[idx_v.at[s]], rows_v.at[b], sems[b]).start()

        def wait_acc(b):
            pltpu.make_async_copy(
                table_hbm.at[idx_v.at[0]], rows_v.at[b], sems[b]).wait()

            def abody(r, _):
                plsc.addupdate(acc_v.at[r, pl.ds(0, LANES)],
                               rows_v[b, r, pl.ds(0, LANES)])
                plsc.addupdate(acc_v.at[r, pl.ds(LANES, LANES)],
                               rows_v[b, r, pl.ds(LANES, LANES)])
                return 0

            lax.fori_loop(0, BPW, abody, 0, unroll=8)

        DEPTH = 8
        for b in range(DEPTH):
            start(b, b)

        def gbody(g, _):
            s0 = DEPTH * g
            for b in range(DEPTH):
                wait_acc(b)
                start(s0 + DEPTH + b, b)
            return 0

        lax.fori_loop(0, SEQ // DEPTH - 1, gbody, 0)
        for b in range(DEPTH):
            wait_acc(b)

        pltpu.sync_copy(acc_v, out_hbm.at[pl.ds(base, BPW)])

    return pool(text, emb_table)


def _mlp(sums, W1, b1, W2, b2):
    BN = 1024

    def mlp_body(x_ref, w1_ref, b1_ref, w2_ref, b2_ref, o_ref):
        x = x_ref[...]
        h = jnp.dot(x, w1_ref[...], preferred_element_type=jnp.float32)
        h = h * (1.0 / SEQ) + b1_ref[...]
        h = jnp.maximum(h, 0.0)
        o_ref[...] = (jnp.dot(h, w2_ref[...], preferred_element_type=jnp.float32)
                      + b2_ref[...])

    return pl.pallas_call(
        mlp_body,
        grid=(BATCH // BN,),
        in_specs=[
            pl.BlockSpec((BN, EMB), lambda i: (i, 0)),
            pl.BlockSpec((EMB, HID), lambda i: (0, 0)),
            pl.BlockSpec((1, HID), lambda i: (0, 0)),
            pl.BlockSpec((HID, OUT), lambda i: (0, 0)),
            pl.BlockSpec((1, OUT), lambda i: (0, 0)),
        ],
        out_specs=pl.BlockSpec((BN, OUT), lambda i: (i, 0)),
        out_shape=jax.ShapeDtypeStruct((BATCH, OUT), jnp.float32),
    )(sums, W1, b1.reshape(1, HID), W2, b2.reshape(1, OUT))


def kernel(text, emb_table, W1, b1, W2, b2):
    text = text.astype(jnp.int32)
    table_rm = _transpose_table(emb_table.T).reshape(VOCAB_P, EMB)
    sums = _pool_sums(text, table_rm)
    return _mlp(sums, W1, b1, W2, b2)


# speedup vs baseline: 1.0048x; 1.0048x over previous
import functools

import jax
import jax.numpy as jnp
from jax import lax
from jax.experimental import pallas as pl
from jax.experimental.pallas import tpu as pltpu
from jax.experimental.pallas import tpu_sc as plsc

EMB = 32
HID = 128
OUT = 2
SEQ = 200
BATCH = 4096

NC = 2
NS = 16
NW = NC * NS
BPW = BATCH // NW
LANES = 16
VPR = EMB // LANES
VECS = BPW * VPR


VOCAB = 1000000
VT = 128
NT = 7813
VOCAB_P = NT * VT
KPW = 246
TILE_ELEMS = VT * EMB


def _transpose_table(table_t):
    mesh = plsc.VectorSubcoreMesh(core_axis_name="c", subcore_axis_name="s")

    @functools.partial(
        pl.kernel,
        mesh=mesh,
        out_type=jax.ShapeDtypeStruct((VOCAB_P * EMB,), jnp.float32),
        scratch_types=[
            pltpu.VMEM((EMB, VT), jnp.float32),
            pltpu.VMEM((EMB, VT), jnp.float32),
            pltpu.VMEM((TILE_ELEMS,), jnp.float32),
            pltpu.VMEM((TILE_ELEMS,), jnp.float32),
            pltpu.SemaphoreType.DMA,
            pltpu.SemaphoreType.DMA,
            pltpu.SemaphoreType.DMA,
            pltpu.SemaphoreType.DMA,
        ],
        compiler_params=pltpu.CompilerParams(
            use_tc_tiling_on_sc=True, needs_layout_passes=False),
    )
    def tr(tt_hbm, out_hbm, src0, src1, dst0, dst1, si0, si1, so0, so1):
        wid = lax.axis_index("s") * NC + lax.axis_index("c")
        t0 = wid * KPW
        srcs = (src0, src1)
        dsts = (dst0, dst1)
        sin = (si0, si1)
        sout = (so0, so1)

        lane = lax.iota(jnp.int32, LANES)

        def voff(k):
            return jnp.minimum(t0 + k, NT - 1) * VT

        def start_in(k, b):
            pltpu.make_async_copy(
                tt_hbm.at[:, pl.ds(voff(k), VT)], srcs[b], sin[b]).start()

        def start_out(k, b):
            pltpu.make_async_copy(
                dsts[b], out_hbm.at[pl.ds(voff(k) * EMB, TILE_ELEMS)],
                sout[b]).start()

        def wait_in(b):
            pltpu.make_async_copy(
                tt_hbm.at[:, pl.ds(0, VT)], srcs[b], sin[b]).wait()

        def wait_out(b):
            pltpu.make_async_copy(
                dsts[b], out_hbm.at[pl.ds(0, TILE_ELEMS)], sout[b]).wait()

        def scatter(b):
            @plsc.parallel_loop(0, EMB * (VT // LANES), unroll=8)
            def _(i):
                d = ((i >> 3) + lane) & (EMB - 1)
                vj = (i & 7) * LANES + lane
                x = plsc.load_gather(srcs[b], [d, vj])
                plsc.store_scatter(dsts[b], [vj * EMB + d], x)

        start_in(0, 0)
        start_in(1, 1)

        def gbody(g, _):
            k0 = 2 * g

            @pl.when(g > 0)
            def _():
                wait_out(0)

            wait_in(0)
            scatter(0)
            start_out(k0, 0)

            @pl.when(k0 + 2 < KPW)
            def _():
                start_in(k0 + 2, 0)

            @pl.when(g > 0)
            def _():
                wait_out(1)

            wait_in(1)
            scatter(1)
            start_out(k0 + 1, 1)

            @pl.when(k0 + 3 < KPW)
            def _():
                start_in(k0 + 3, 1)

            return 0

        lax.fori_loop(0, KPW // 2, gbody, 0)
        wait_out(0)
        wait_out(1)

    return tr(table_t)


def _pool_sums(text, emb_table):
    mesh = plsc.VectorSubcoreMesh(core_axis_name="c", subcore_axis_name="s")

    @functools.partial(
        pl.kernel,
        mesh=mesh,
        out_type=jax.ShapeDtypeStruct((BATCH, EMB), jnp.float32),
        scratch_types=[
            pltpu.VMEM((SEQ, BPW), jnp.int32),
            pltpu.VMEM((BPW, EMB), jnp.float32),
            pltpu.VMEM((4, BPW, EMB), jnp.float32),
        ] + [pltpu.SemaphoreType.DMA] * 4,
        compiler_params=pltpu.CompilerParams(use_tc_tiling_on_sc=False),
    )
    def pool(text_hbm, table_hbm, out_hbm, idx_v, acc_v, rows_v, *sems):
        wid = lax.axis_index("s") * NC + lax.axis_index("c")
        base = wid * BPW
        pltpu.sync_copy(text_hbm.at[:, pl.ds(base, BPW)], idx_v)

        zero = jnp.zeros((LANES,), jnp.float32)

        def zbody(r, _):
            acc_v[r, pl.ds(0, LANES)] = zero
            acc_v[r, pl.ds(LANES, LANES)] = zero
            return 0

        lax.fori_loop(0, BPW, zbody, 0, unroll=8)

        def start(s, b):
            pltpu.make_async_copy(
                table_hbm.at[idx_v.at[s]], rows_v.at[b], sems[b]).start()

        def wait_acc(b):
            pltpu.make_async_copy(
                table_hbm.at[idx_v.at[0]], rows_v.at[b], sems[b]).wait()

            def abody(r, _):
                plsc.addupdate(acc_v.at[r, pl.ds(0, LANES)],
                               rows_v[b, r, pl.ds(0, LANES)])
                plsc.addupdate(acc_v.at[r, pl.ds(LANES, LANES)],
                               rows_v[b, r, pl.ds(LANES, LANES)])
                return 0

            lax.fori_loop(0, BPW, abody, 0, unroll=8)

        DEPTH = 4
        for b in range(DEPTH):
            start(b, b)

        def gbody(g, _):
            s0 = DEPTH * g
            for b in range(DEPTH):
                wait_acc(b)
                start(s0 + DEPTH + b, b)
            return 0

        lax.fori_loop(0, SEQ // DEPTH - 1, gbody, 0)
        for b in range(DEPTH):
            wait_acc(b)

        pltpu.sync_copy(acc_v, out_hbm.at[pl.ds(base, BPW)])

    return pool(text, emb_table)


def _mlp(sums, W1, b1, W2, b2):
    BN = 1024

    def mlp_body(x_ref, w1_ref, b1_ref, w2_ref, b2_ref, o_ref):
        x = x_ref[...]
        h = jnp.dot(x, w1_ref[...], preferred_element_type=jnp.float32)
        h = h * (1.0 / SEQ) + b1_ref[...]
        h = jnp.maximum(h, 0.0)
        o_ref[...] = (jnp.dot(h, w2_ref[...], preferred_element_type=jnp.float32)
                      + b2_ref[...])

    return pl.pallas_call(
        mlp_body,
        grid=(BATCH // BN,),
        in_specs=[
            pl.BlockSpec((BN, EMB), lambda i: (i, 0)),
            pl.BlockSpec((EMB, HID), lambda i: (0, 0)),
            pl.BlockSpec((1, HID), lambda i: (0, 0)),
            pl.BlockSpec((HID, OUT), lambda i: (0, 0)),
            pl.BlockSpec((1, OUT), lambda i: (0, 0)),
        ],
        out_specs=pl.BlockSpec((BN, OUT), lambda i: (i, 0)),
        out_shape=jax.ShapeDtypeStruct((BATCH, OUT), jnp.float32),
    )(sums, W1, b1.reshape(1, HID), W2, b2.reshape(1, OUT))


def kernel(text, emb_table, W1, b1, W2, b2):
    text = text.astype(jnp.int32)
    table_rm = _transpose_table(emb_table.T).reshape(VOCAB_P, EMB)
    sums = _pool_sums(text, table_rm)
    return _mlp(sums, W1, b1, W2, b2)
